# Initial kernel scaffold; baseline (speedup 1.0000x reference)
import functools

import jax
import jax.numpy as jnp
from jax import lax
from jax.experimental import pallas as pl
from jax.experimental.pallas import tpu as pltpu
from jax.experimental.pallas import tpu_sc as plsc

B, L, C = 4096, 20, 16
D = 32
CHAR_VOC = 1000

_info = plsc.get_sparse_core_info()
NC, NS, LANES = _info.num_cores, _info.num_subcores, _info.num_lanes
NW = NC * NS
RPW = B // NW
CH = 4
NCHUNK = RPW // CH

_mesh = plsc.VectorSubcoreMesh(core_axis_name="c", subcore_axis_name="s")


@functools.partial(
    pl.kernel,
    out_type=jax.ShapeDtypeStruct((B, 2 * D), jnp.float32),
    mesh=_mesh,
    scratch_types=[
        pltpu.VMEM((CHAR_VOC, D), jnp.float32),
        pltpu.VMEM((NCHUNK, CH * L), jnp.int32),
        pltpu.VMEM((RPW * L * C,), jnp.int32),
        pltpu.VMEM((CH * L, D), jnp.float32),
        pltpu.VMEM((RPW, 2 * D), jnp.float32),
        pltpu.SemaphoreType.DMA,
    ],
)
def _emb_kernel(xw_hbm, xc_hbm, wt_hbm, ct_hbm, out_hbm,
                ct_v, wi_v, ci_v, wr_v, out_v, sem):
    wid = lax.axis_index("s") * NC + lax.axis_index("c")
    pltpu.sync_copy(ct_hbm, ct_v)
    pltpu.sync_copy(xw_hbm.at[wid], wi_v)
    pltpu.sync_copy(xc_hbm.at[wid], ci_v)

    wscale = jnp.float32(1.0 / L)
    cscale = jnp.float32(1.0 / (L * C))
    zero = jnp.zeros((LANES,), jnp.float32)

    def chunk_body(c, carry):
        pltpu.async_copy(wt_hbm.at[wi_v.at[c]], wr_v, sem).wait()
        for r in range(CH):
            row = c * CH + r

            def wbody(j, acc):
                a0, a1 = acc
                a0 = a0 + wr_v[r * L + j, pl.ds(0, LANES)]
                a1 = a1 + wr_v[r * L + j, pl.ds(LANES, LANES)]
                return (a0, a1)

            w0, w1 = lax.fori_loop(0, L, wbody, (zero, zero))

            cbase = row * (L * C)

            def cbody(j, acc):
                a0, a1 = acc
                s = ci_v[cbase + j]
                a0 = a0 + ct_v[s, pl.ds(0, LANES)]
                a1 = a1 + ct_v[s, pl.ds(LANES, LANES)]
                return (a0, a1)

            c0, c1 = lax.fori_loop(0, L * C, cbody, (zero, zero))

            out_v[row, pl.ds(0, LANES)] = w0 * wscale
            out_v[row, pl.ds(LANES, LANES)] = w1 * wscale
            out_v[row, pl.ds(2 * LANES, LANES)] = c0 * cscale
            out_v[row, pl.ds(3 * LANES, LANES)] = c1 * cscale
        return carry

    lax.fori_loop(0, NCHUNK, chunk_body, 0)
    pltpu.sync_copy(out_v, out_hbm.at[pl.ds(wid * RPW, RPW)])


def kernel(x, word_pos, x_char, x_mask, word_table, char_table):
    xw = x.reshape(NW, NCHUNK, CH * L)
    xc = x_char.reshape(NW, RPW * L * C)
    return _emb_kernel(xw, xc, word_table, char_table)


# speedup vs baseline: 9.1170x; 9.1170x over previous
import functools

import jax
import jax.numpy as jnp
from jax import lax
from jax.experimental import pallas as pl
from jax.experimental.pallas import tpu as pltpu
from jax.experimental.pallas import tpu_sc as plsc

B, L, C = 4096, 20, 16
D = 32
CHAR_VOC = 1000

_info = plsc.get_sparse_core_info()
NC, NS, LANES = _info.num_cores, _info.num_subcores, _info.num_lanes
NW = NC * NS
RPW = B // NW
CH = 4
NCHUNK = RPW // CH

_mesh = plsc.VectorSubcoreMesh(core_axis_name="c", subcore_axis_name="s")


@functools.partial(
    pl.kernel,
    out_type=jax.ShapeDtypeStruct((B, 2 * D), jnp.float32),
    mesh=_mesh,
    compiler_params=pltpu.CompilerParams(use_tc_tiling_on_sc=False),
    scratch_types=[
        pltpu.VMEM((CHAR_VOC, D), jnp.float32),
        pltpu.VMEM((NCHUNK, CH * L), jnp.int32),
        pltpu.VMEM((RPW * L * C,), jnp.int32),
        pltpu.VMEM((CH * L, D), jnp.float32),
        pltpu.VMEM((RPW, 2 * D), jnp.float32),
        pltpu.SemaphoreType.DMA,
    ],
)
def _emb_kernel(xw_hbm, xc_hbm, wt_hbm, ct_hbm, out_hbm,
                ct_v, wi_v, ci_v, wr_v, out_v, sem):
    wid = lax.axis_index("s") * NC + lax.axis_index("c")
    pltpu.sync_copy(ct_hbm, ct_v)
    pltpu.sync_copy(xw_hbm.at[wid], wi_v)
    pltpu.sync_copy(xc_hbm.at[wid], ci_v)

    wscale = jnp.float32(1.0 / L)
    cscale = jnp.float32(1.0 / (L * C))
    zero = jnp.zeros((LANES,), jnp.float32)

    def chunk_body(c, carry):
        pltpu.async_copy(wt_hbm.at[wi_v.at[c]], wr_v, sem).wait()
        for r in range(CH):
            row = c * CH + r

            def wbody(j, acc):
                a0, a1 = acc
                a0 = a0 + wr_v[r * L + j, pl.ds(0, LANES)]
                a1 = a1 + wr_v[r * L + j, pl.ds(LANES, LANES)]
                return (a0, a1)

            w0, w1 = lax.fori_loop(0, L, wbody, (zero, zero))

            cbase = row * (L * C)

            def cbody(k, acc):
                a0, a1 = acc
                idxv = ci_v[pl.ds(cbase + k * LANES, LANES)]
                for t in range(LANES):
                    s = idxv[t]
                    a0 = a0 + ct_v[s, pl.ds(0, LANES)]
                    a1 = a1 + ct_v[s, pl.ds(LANES, LANES)]
                return (a0, a1)

            c0, c1 = lax.fori_loop(0, (L * C) // LANES, cbody, (zero, zero))

            out_v[row, pl.ds(0, LANES)] = w0 * wscale
            out_v[row, pl.ds(LANES, LANES)] = w1 * wscale
            out_v[row, pl.ds(2 * LANES, LANES)] = c0 * cscale
            out_v[row, pl.ds(3 * LANES, LANES)] = c1 * cscale
        return carry

    lax.fori_loop(0, NCHUNK, chunk_body, 0)
    pltpu.sync_copy(out_v, out_hbm.at[pl.ds(wid * RPW, RPW)])


def kernel(x, word_pos, x_char, x_mask, word_table, char_table):
    xw = x.reshape(NW, NCHUNK, CH * L)
    xc = x_char.reshape(NW, RPW * L * C)
    return _emb_kernel(xw, xc, word_table, char_table)


# speedup vs baseline: 9.5362x; 1.0460x over previous
import functools

import jax
import jax.numpy as jnp
from jax import lax
from jax.experimental import pallas as pl
from jax.experimental.pallas import tpu as pltpu
from jax.experimental.pallas import tpu_sc as plsc

B, L, C = 4096, 20, 16
D = 32
CHAR_VOC = 1000

_info = plsc.get_sparse_core_info()
NC, NS, LANES = _info.num_cores, _info.num_subcores, _info.num_lanes
NW = NC * NS
RPW = B // NW
CH = 4
NCHUNK = RPW // CH
CW = CH * L * C

_mesh = plsc.VectorSubcoreMesh(core_axis_name="c", subcore_axis_name="s")


@functools.partial(
    pl.kernel,
    out_type=jax.ShapeDtypeStruct((B, 2 * D), jnp.float32),
    mesh=_mesh,
    compiler_params=pltpu.CompilerParams(use_tc_tiling_on_sc=False),
    scratch_types=[
        pltpu.VMEM((CHAR_VOC, LANES), jnp.uint32),
        pltpu.VMEM((NCHUNK, CH * L), jnp.int32),
        pltpu.VMEM((RPW * L * C,), jnp.int32),
        pltpu.VMEM((2, CH * L, D), jnp.float32),
        pltpu.VMEM((RPW, 2 * D), jnp.float32),
        pltpu.SemaphoreType.DMA,
        pltpu.SemaphoreType.DMA,
    ],
)
def _emb_kernel(xw_hbm, xc_hbm, wt_hbm, ctp_hbm, out_hbm,
                ct_v, wi_v, ci_v, wr_v, out_v, sem0, sem1):
    wid = lax.axis_index("s") * NC + lax.axis_index("c")
    pltpu.sync_copy(ctp_hbm, ct_v)
    pltpu.sync_copy(xw_hbm.at[wid], wi_v)
    pltpu.sync_copy(xc_hbm.at[wid], ci_v)

    wscale = jnp.float32(1.0 / L)
    cscale = jnp.float32(1.0 / (L * C))
    zero = jnp.zeros((LANES,), jnp.float32)
    himask = jnp.full((LANES,), 0xFFFF0000, jnp.uint32)
    sh16 = jnp.uint32(16)

    sems = (sem0, sem1)
    pltpu.async_copy(wt_hbm.at[wi_v.at[0]], wr_v.at[0], sem0)
    pltpu.async_copy(wt_hbm.at[wi_v.at[1]], wr_v.at[1], sem1)

    def process_chunk(c, buf):
        sem = sems[buf]
        pltpu.make_async_copy(wt_hbm.at[pl.ds(0, CH * L)], wr_v.at[buf], sem).wait()
        for r in range(CH):
            row = c * CH + r

            def wbody(j, acc):
                a0, a1, a2, a3 = acc
                b = r * L + j * 2
                a0 = a0 + wr_v[buf, b, pl.ds(0, LANES)]
                a1 = a1 + wr_v[buf, b, pl.ds(LANES, LANES)]
                a2 = a2 + wr_v[buf, b + 1, pl.ds(0, LANES)]
                a3 = a3 + wr_v[buf, b + 1, pl.ds(LANES, LANES)]
                return (a0, a1, a2, a3)

            w = lax.fori_loop(0, L // 2, wbody, (zero, zero, zero, zero))
            w0 = w[0] + w[2]
            w1 = w[1] + w[3]

            def cbody(k, acc):
                lo0, hi0, lo1, hi1 = acc
                idxv = ci_v[pl.ds(c * CW + r * (L * C) + k * LANES, LANES)]
                for t in range(LANES):
                    s = idxv[t]
                    u = ct_v[s]
                    lov = lax.bitcast_convert_type(u << sh16, jnp.float32)
                    hiv = lax.bitcast_convert_type(u & himask, jnp.float32)
                    if t % 2 == 0:
                        lo0 = lo0 + lov
                        hi0 = hi0 + hiv
                    else:
                        lo1 = lo1 + lov
                        hi1 = hi1 + hiv
                return (lo0, hi0, lo1, hi1)

            a = lax.fori_loop(0, (L * C) // LANES, cbody, (zero, zero, zero, zero))
            c0 = a[0] + a[2]
            c1 = a[1] + a[3]

            out_v[row, pl.ds(0, LANES)] = w0 * wscale
            out_v[row, pl.ds(LANES, LANES)] = w1 * wscale
            out_v[row, pl.ds(2 * LANES, LANES)] = c0 * cscale
            out_v[row, pl.ds(3 * LANES, LANES)] = c1 * cscale
        @pl.when(c + 2 < NCHUNK)
        def _():
            pltpu.async_copy(wt_hbm.at[wi_v.at[c + 2]], wr_v.at[buf], sem)

    def outer(i, carry):
        process_chunk(2 * i, 0)
        process_chunk(2 * i + 1, 1)
        return carry

    lax.fori_loop(0, NCHUNK // 2, outer, 0)
    pltpu.sync_copy(out_v, out_hbm.at[pl.ds(wid * RPW, RPW)])


def kernel(x, word_pos, x_char, x_mask, word_table, char_table):
    xw = x.reshape(NW, NCHUNK, CH * L)
    xc = x_char.reshape(NW, RPW * L * C)
    cb = char_table.astype(jnp.bfloat16)
    lo = lax.bitcast_convert_type(cb[:, :LANES], jnp.uint16).astype(jnp.uint32)
    hi = lax.bitcast_convert_type(cb[:, LANES:], jnp.uint16).astype(jnp.uint32)
    ctp = lo | (hi << 16)
    return _emb_kernel(xw, xc, word_table, ctp)
